# Initial kernel scaffold; baseline (speedup 1.0000x reference)
import jax
import jax.numpy as jnp
from jax.experimental import pallas as pl


def kernel(x, enc_feat, km1_in, km1_out, km2_in, km2_out, km3_in, km3_out, W1, W2, W3, Wp, bp, g1, b1, g2, b2, g3, b3):
    raise NotImplementedError("write your pallas kernel here")


# speedup vs baseline: 4.3058x; 4.3058x over previous
import functools

import jax
import jax.numpy as jnp
from jax import lax
from jax.experimental import pallas as pl
from jax.experimental.pallas import tpu as pltpu
from jax.experimental.pallas import tpu_sc as plsc

N_CORES = 2
N_SUB = 16
N_WORKERS = N_CORES * N_SUB
ROW_GRP = 128


def _gemm_batch(xin, W):
    K, Cin, Cout = W.shape
    N = xin.shape[0]
    BN = 2048
    nj = N // BN

    def body(x_ref, w_ref, o_ref):
        o_ref[0] = jnp.dot(x_ref[...], w_ref[0],
                           preferred_element_type=jnp.float32)

    return pl.pallas_call(
        body,
        grid=(nj, K),
        in_specs=[
            pl.BlockSpec((BN, Cin), lambda j, k: (j, 0)),
            pl.BlockSpec((1, Cin, Cout), lambda j, k: (k, 0, 0)),
        ],
        out_specs=pl.BlockSpec((1, BN, Cout), lambda j, k: (k, j, 0)),
        out_shape=jax.ShapeDtypeStruct((K, N, Cout), jnp.float32),
    )(xin, W)


def _sc_gather_scatter_add(z_flat, gidx, sidx, n_out, n_k):
    C = z_flat.shape[1]
    M = gidx.shape[0]
    per_w = M // N_WORKERS
    grp_rows = n_k * ROW_GRP
    n_chunks = per_w // grp_rows
    assert n_chunks * grp_rows == per_w
    gidx2 = gidx.reshape(M // ROW_GRP, 1, ROW_GRP)
    sidx2 = sidx.reshape(M // ROW_GRP, 1, ROW_GRP)
    rows_per_sub = n_out // N_SUB

    mesh = plsc.VectorSubcoreMesh(core_axis_name="c", subcore_axis_name="s")

    @functools.partial(
        pl.kernel,
        out_type=jax.ShapeDtypeStruct((N_CORES * n_out, C), jnp.float32),
        mesh=mesh,
        scratch_types=[
            pltpu.VMEM((n_k, 1, ROW_GRP), jnp.int32),
            pltpu.VMEM((n_k, 1, ROW_GRP), jnp.int32),
            pltpu.VMEM((grp_rows, C), jnp.float32),
            pltpu.VMEM_SHARED((n_out, C), jnp.float32),
            pltpu.SemaphoreType.DMA,
        ],
        compiler_params=pltpu.CompilerParams(use_tc_tiling_on_sc=False),
    )
    def scatter_kernel(z_hbm, gidx_hbm, sidx_hbm, out_hbm,
                       gidx_v, sidx_v, rows_v, acc_sh, sem):
        cid = lax.axis_index("c")
        sid = lax.axis_index("s")
        wid = sid * N_CORES + cid

        zero16 = jnp.zeros((16,), jnp.float32)

        def zrow(r, carry):
            for c in range(C // 16):
                rows_v[r, pl.ds(c * 16, 16)] = zero16
            return carry

        zrows = min(512, grp_rows)
        lax.fori_loop(0, zrows, zrow, 0)
        for z in range(rows_per_sub // zrows):
            pltpu.sync_copy(
                rows_v.at[pl.ds(0, zrows)],
                acc_sh.at[pl.ds(sid * rows_per_sub + z * zrows, zrows)],
            )
        plsc.subcore_barrier()

        def body(i, carry):
            grp0 = (wid * per_w + i * grp_rows) // ROW_GRP
            pltpu.sync_copy(gidx_hbm.at[pl.ds(grp0, n_k)], gidx_v)
            pltpu.sync_copy(sidx_hbm.at[pl.ds(grp0, n_k)], sidx_v)
            cps = [
                pltpu.async_copy(z_hbm.at[gidx_v.at[j, 0]],
                                 rows_v.at[pl.ds(j * ROW_GRP, ROW_GRP)], sem)
                for j in range(n_k)
            ]
            for cp in cps:
                cp.wait()
            for j in range(n_k):
                pltpu.sync_copy(rows_v.at[pl.ds(j * ROW_GRP, ROW_GRP)],
                                acc_sh.at[sidx_v.at[j, 0]], add=True)
            return carry

        lax.fori_loop(0, n_chunks, body, 0)

        plsc.subcore_barrier()
        pltpu.sync_copy(
            acc_sh.at[pl.ds(sid * rows_per_sub, rows_per_sub)],
            out_hbm.at[pl.ds(cid * n_out + sid * rows_per_sub, rows_per_sub)],
        )

    return scatter_kernel(z_flat, gidx2, sidx2)


LOG2E = 1.4426950408889634
LN2_HI = 0.693359375
LN2_LO = -2.12194440e-4


def _exp_accurate(y):
    yc = jnp.maximum(y, -80.0)
    k = jnp.floor(yc * LOG2E + 0.5)
    r = (yc - k * LN2_HI) - k * LN2_LO
    p = 1.0 + r * (1.0 + r * (0.5 + r * (
        0.16666666666666666 + r * (0.041666666666666664 + r * (
            0.008333333333333333 + r * 0.001388888888888889)))))
    two_k = jax.lax.bitcast_convert_type(
        (k.astype(jnp.int32) + 127) << 23, jnp.float32)
    return p * two_k


def _expm1_accurate(y):
    t = y * (1.0 + y * (0.5 + y * (0.16666666666666666 + y * (
        0.041666666666666664 + y * (0.008333333333333333 + y * (
            0.001388888888888889 + y * 0.0001984126984126984))))))
    return jnp.where(y > -0.34657359, t, _exp_accurate(y) - 1.0)


def _rsqrt_accurate(u):
    s = lax.rsqrt(u)
    s = s * (1.5 - 0.5 * u * s * s)
    s = s * (1.5 - 0.5 * u * s * s)
    return s


def _sum_partials(parts):
    two_n, C = parts.shape
    n = two_n // 2

    def body(p_ref, o_ref):
        o_ref[...] = p_ref[0:n, :] + p_ref[n:two_n, :]

    return pl.pallas_call(
        body, out_shape=jax.ShapeDtypeStruct((n, C), jnp.float32),
    )(parts)


def _stats(h):
    m = jnp.mean(h, axis=0, keepdims=True)
    v = jnp.var(h, axis=0, keepdims=True)
    return m, jnp.sqrt(v + 1e-5)


def _norm_elu(parts, g, b, enc=None):
    two_n, C = parts.shape
    n = two_n // 2
    h = _sum_partials(parts)
    m, rec = _stats(h)

    def body(h_ref, m_ref, r_ref, g_ref, b_ref, *rest):
        o_ref = rest[-1]
        y = (h_ref[...] - m_ref[...]) / r_ref[...] * g_ref[...] + b_ref[...]
        y = jnp.where(y > 0.0, y, _expm1_accurate(jnp.minimum(y, 0.0)))
        if enc is not None:
            y = y + rest[0][...]
        o_ref[...] = y

    args = [h, m, rec, g.reshape(1, C), b.reshape(1, C)]
    if enc is not None:
        args.append(enc)
    return pl.pallas_call(
        body,
        out_shape=jax.ShapeDtypeStruct((n, C), jnp.float32),
    )(*args)


def _norm_elu_prune(parts, g, b, Wp, bp):
    two_n, C = parts.shape
    n = two_n // 2
    h = _sum_partials(parts)
    m, rec = _stats(h)

    def body(h_ref, m_ref, r_ref, g_ref, b_ref, wp_ref, bp_ref, o_ref):
        y = (h_ref[...] - m_ref[...]) / r_ref[...] * g_ref[...] + b_ref[...]
        y = jnp.where(y > 0.0, y, _expm1_accurate(jnp.minimum(y, 0.0)))
        pr_full = jnp.dot(y, wp_ref[...], preferred_element_type=jnp.float32)
        pr = pr_full[:, 0:1] + bp_ref[...]
        o_ref[...] = jnp.where(pr > 0.0, y, 0.0)

    wp_pad = jnp.pad(Wp, ((0, 0), (0, 127)))
    return pl.pallas_call(
        body,
        out_shape=jax.ShapeDtypeStruct((n, C), jnp.float32),
    )(h, m, rec, g.reshape(1, C), b.reshape(1, C), wp_pad,
      bp.reshape(1, 1))


def _sparse_conv_layer(h, W, km_in, km_out, n_out, n_k):
    K, N = km_in.shape
    Z = _gemm_batch(h, W)
    gidx = (km_in + jnp.arange(K, dtype=jnp.int32)[:, None] * N).reshape(-1)
    return _sc_gather_scatter_add(Z.reshape(K * N, W.shape[2]), gidx,
                                  km_out.reshape(-1), n_out, n_k)


def kernel(x, enc_feat, km1_in, km1_out, km2_in, km2_out, km3_in, km3_out,
           W1, W2, W3, Wp, bp, g1, b1, g2, b2, g3, b3):
    n_out = enc_feat.shape[0]
    p1 = _sparse_conv_layer(x, W1, km1_in, km1_out, n_out, n_k=6)
    h1 = _norm_elu(p1, g1, b1)
    p2 = _sparse_conv_layer(h1, W2, km2_in, km2_out, n_out, n_k=6)
    h2 = _norm_elu(p2, g2, b2, enc=enc_feat)
    p3 = _sparse_conv_layer(h2, W3, km3_in, km3_out, n_out, n_k=4)
    return _norm_elu_prune(p3, g3, b3, Wp, bp)
